# baseline (speedup 1.0000x reference)
import functools

import jax
import jax.numpy as jnp
from jax import lax
from jax.experimental import pallas as pl
from jax.experimental.pallas import tpu as pltpu
from jax.experimental.pallas import tpu_sc as plsc

F32 = jnp.float32
BF16 = jnp.bfloat16
H = 128
DE = 16
N_MAIN = 10000
N_SUPE = 1000
E_MAIN = 320000
E_WHOLE = 40000
NCORES = 2
NSUB = 16
NW = NCORES * NSUB
CB = 80
H2 = H // 2
WD = 144
NCH_B = E_MAIN // CB

import numpy as _np
_PERM = _np.empty(H, _np.int32)
for _j in range(H // 32):
    _PERM[32 * _j:32 * _j + 16] = 32 * _j + 2 * _np.arange(16)
    _PERM[32 * _j + 16:32 * _j + 32] = 32 * _j + 1 + 2 * _np.arange(16)


def _mesh():
    return plsc.VectorSubcoreMesh(core_axis_name="c", subcore_axis_name="s",
                                  num_cores=NCORES, num_subcores=NSUB)


def _pre_main_body(mf, waT, ba, scmT, bscm, scsT, bscs, attw, w2aT, bu2,
                   a_out, mp_out, tmw_out, proj_out):
    x = mf[...]
    a = jnp.tanh(jnp.dot(x, waT[...], preferred_element_type=F32) + ba[...])
    a_out[...] = a.astype(BF16)
    mp_out[...] = (jnp.dot(a, scmT[...], preferred_element_type=F32)
                   + bscm[...]).astype(BF16)
    tmw_out[...] = ((jnp.dot(a, scsT[...], preferred_element_type=F32)
                     + bscs[...]) * attw[...]).astype(BF16)
    proj_out[...] = (jnp.dot(x, w2aT[...], preferred_element_type=F32)
                     + bu2[...]).astype(BF16)


def _pre_supe_body(sf, waT, ba, scmT, bscm, scsT, bscs, attw,
                   a_out, smw_out, sp_out):
    a = jnp.tanh(jnp.dot(sf[...], waT[...], preferred_element_type=F32) + ba[...])
    a_out[...] = a.astype(BF16)
    smw_out[...] = ((jnp.dot(a, scmT[...], preferred_element_type=F32)
                     + bscm[...]) * attw[...]).astype(BF16)
    sp_out[...] = (jnp.dot(a, scsT[...], preferred_element_type=F32)
                   + bscs[...]).astype(BF16)


def _ep_body(ea, w2bT, out):
    out[...] = jnp.dot(ea[...], w2bT[...],
                       preferred_element_type=F32).astype(BF16)


def _full(shape):
    return pl.BlockSpec(shape, lambda i: tuple(0 for _ in shape))


def _tc_pre_main(mf, waT, ba, scmT, bscm, scsT, bscs, attw, w2aT, bu2):
    nb = 5
    blk = N_MAIN // nb
    row = pl.BlockSpec((blk, H), lambda i: (i, 0))
    w = _full((H, H))
    b = _full((1, H))
    out = jax.ShapeDtypeStruct((N_MAIN, H), BF16)
    return pl.pallas_call(
        _pre_main_body,
        grid=(nb,),
        in_specs=[row, w, b, w, b, w, b, b, w, b],
        out_specs=[row, row, row, row],
        out_shape=[out, out, out, out],
    )(mf, waT, ba, scmT, bscm, scsT, bscs, attw, w2aT, bu2)


def _tc_pre_supe(sf, waT, ba, scmT, bscm, scsT, bscs, attw):
    out = jax.ShapeDtypeStruct((N_SUPE, H), BF16)
    return pl.pallas_call(
        _pre_supe_body,
        out_shape=[out, out, out],
    )(sf, waT, ba, scmT, bscm, scsT, bscs, attw)


def _tc_ep(ea, w2bT):
    nb = 40
    blk = E_MAIN // nb
    return pl.pallas_call(
        _ep_body,
        grid=(nb,),
        in_specs=[pl.BlockSpec((blk, DE), lambda i: (i, 0)), _full((DE, H))],
        out_specs=pl.BlockSpec((blk, H), lambda i: (i, 0)),
        out_shape=jax.ShapeDtypeStruct((E_MAIN, H), BF16),
    )(ea, w2bT)


def _sc_attn_body(nseg, ca, seg_hbm, oth_hbm, w_tab, x_tab, v_tab,
                  outp, num_sh, *bufs):
    (seg0, oth0, bw0, bx0, bv0, ch0, sg0, ss0,
     seg1, oth1, bw1, bx1, bv1, ch1, sg1, ss1) = bufs
    B = ((seg0, oth0, bw0, bx0, bv0, ch0, sg0, ss0),
         (seg1, oth1, bw1, bx1, bv1, ch1, sg1, ss1))
    c = lax.axis_index("c")
    s = lax.axis_index("s")
    wid = s * NCORES + c
    nch_tot = E_WHOLE // ca

    def zrow(r, carry):
        for j in range(WD // 16):
            ch0[r, pl.ds(j * 16, 16)] = jnp.zeros((16,), F32)
        return carry
    lax.fori_loop(0, min(ca, 64), zrow, 0)

    zc = min(ca, 64)
    nzc = ((nseg + zc - 1) // zc - s + NSUB - 1) // NSUB

    def z(i, carry):
        b = jnp.minimum((s + NSUB * i) * zc, nseg - zc)
        pltpu.sync_copy(ch0.at[pl.ds(0, zc)], num_sh.at[pl.ds(b, zc)])
        return carry
    lax.fori_loop(0, nzc, z, 0)
    plsc.subcore_barrier()

    lane = lax.iota(jnp.int32, 16)
    m0 = lane == 0
    perms = [(lane ^ k).reshape(16, 1) for k in (8, 4, 2, 1)]
    gdn = lax.GatherDimensionNumbers(
        offset_dims=(), collapsed_slice_dims=(0,), start_index_map=(0,))

    def allsum(v):
        for pm in perms:
            v = v + lax.gather(v, pm, gdn, (1,),
                               mode=lax.GatherScatterMode.PROMISE_IN_BOUNDS)
        return v

    nch = (nch_tot - wid + NW - 1) // NW

    def issue_g(i, p):
        (sv, ov, bw, bx, bv, _, sg, _) = B[p]
        base = (wid + NW * i) * ca
        pltpu.sync_copy(seg_hbm.at[pl.ds(base, ca)], sv)
        pltpu.sync_copy(oth_hbm.at[pl.ds(base, ca)], ov)
        pltpu.async_copy(w_tab.at[sv], bw, sg)
        pltpu.async_copy(x_tab.at[ov], bx, sg)
        pltpu.async_copy(v_tab.at[ov], bv, sg)

    def wait_g(p):
        (sv, ov, bw, bx, bv, _, sg, _) = B[p]
        pltpu.make_async_copy(w_tab.at[sv], bw, sg).wait()
        pltpu.make_async_copy(x_tab.at[ov], bx, sg).wait()
        pltpu.make_async_copy(v_tab.at[ov], bv, sg).wait()

    def start_s(p):
        (sv, _, _, _, _, ch, _, ss) = B[p]
        pltpu.async_copy(ch, num_sh.at[sv], ss, add=True)

    def wait_s(p):
        (sv, _, _, _, _, ch, _, ss) = B[p]
        pltpu.make_async_copy(ch, num_sh.at[sv], ss).wait()

    def compute(p):
        (_, _, bw, bx, bv, ch, _, _) = B[p]

        def edge(e, ecarry):
            acc = jnp.zeros((16,), F32)
            for j in range(H2 // 16):
                sl = pl.ds(j * 16, 16)
                wa, wb = plsc.unpack(plsc.bitcast(bw[e, sl], BF16),
                                     format=plsc.PackFormat.INTERLEAVED)
                xa, xb = plsc.unpack(plsc.bitcast(bx[e, sl], BF16),
                                     format=plsc.PackFormat.INTERLEAVED)
                acc = acc + wa * xa + wb * xb
            ev = jnp.exp(allsum(acc))
            for j in range(H2 // 16):
                sl = pl.ds(j * 16, 16)
                va, vb = plsc.unpack(plsc.bitcast(bv[e, sl], BF16),
                                     format=plsc.PackFormat.INTERLEAVED)
                ch[e, pl.ds(2 * j * 16, 16)] = ev * va
                ch[e, pl.ds((2 * j + 1) * 16, 16)] = ev * vb
            ch[e, pl.ds(H, 16)] = jnp.where(m0, ev, 0.0)
            return ecarry
        lax.fori_loop(0, ca, edge, 0)

    pl.when(nch > 0)(lambda: issue_g(0, 0))
    pl.when(nch > 1)(lambda: issue_g(1, 1))

    def pair(g, carry):
        i0 = 2 * g

        def b0():
            wait_g(0)
            compute(0)
            start_s(0)
        pl.when(i0 < nch)(b0)

        def a0():
            wait_s(0)
            issue_g(i0 + 2, 0)
        pl.when(i0 + 2 < nch)(a0)

        def b1():
            wait_g(1)
            compute(1)
            start_s(1)
        pl.when(i0 + 1 < nch)(b1)

        def a1():
            wait_s(1)
            issue_g(i0 + 3, 1)
        pl.when(i0 + 3 < nch)(a1)
        return carry
    lax.fori_loop(0, (nch + 1) // 2, pair, 0)
    pl.when(nch >= 1)(lambda: wait_s(0))
    pl.when(nch >= 2)(lambda: wait_s(1))
    plsc.subcore_barrier()

    def wb(i, carry):
        b = jnp.minimum((s + NSUB * i) * zc, nseg - zc)
        pltpu.sync_copy(num_sh.at[pl.ds(b, zc)], outp.at[c, pl.ds(b, zc)])
        return carry
    lax.fori_loop(0, nzc, wb, 0)


def _sc_attn_side(nseg, ca, seg_idx, oth_idx, w_tab, x_tab, v_tab):
    buf = [
        pltpu.VMEM((ca,), jnp.int32),
        pltpu.VMEM((ca,), jnp.int32),
        pltpu.VMEM((ca, H2), jnp.int32),
        pltpu.VMEM((ca, H2), jnp.int32),
        pltpu.VMEM((ca, H2), jnp.int32),
        pltpu.VMEM((ca, WD), F32),
        pltpu.SemaphoreType.DMA,
        pltpu.SemaphoreType.DMA,
    ]
    return pl.kernel(
        functools.partial(_sc_attn_body, nseg, ca),
        out_type=jax.ShapeDtypeStruct((NCORES, nseg, WD), F32),
        mesh=_mesh(),
        compiler_params=pltpu.CompilerParams(use_tc_tiling_on_sc=False,
                                             needs_layout_passes=False),
        scratch_types=[pltpu.VMEM_SHARED((nseg, WD), F32)] + buf + buf,
    )(seg_idx, oth_idx, w_tab, x_tab, v_tab)


def _sc_wln_body(sidx_hbm, didx_hbm, proj, ep, aggp, agg_sh, *bufs):
    (si0, di0, pr0, ep0, ch0, sg0, ss0,
     si1, di1, pr1, ep1, ch1, sg1, ss1) = bufs
    B = ((si0, di0, pr0, ep0, ch0, sg0, ss0),
         (si1, di1, pr1, ep1, ch1, sg1, ss1))
    c = lax.axis_index("c")
    s = lax.axis_index("s")
    wid = s * NCORES + c

    def zrow(r, carry):
        for j in range(H // 16):
            ch0[r, pl.ds(j * 16, 16)] = jnp.zeros((16,), F32)
        return carry
    lax.fori_loop(0, CB, zrow, 0)

    nz = ((N_MAIN + CB - 1) // CB - s + NSUB - 1) // NSUB

    def z(i, carry):
        b = jnp.minimum((s + NSUB * i) * CB, N_MAIN - CB)
        pltpu.sync_copy(ch0, agg_sh.at[pl.ds(b, CB)])
        return carry
    lax.fori_loop(0, nz, z, 0)
    plsc.subcore_barrier()

    nch = (NCH_B - wid + NW - 1) // NW

    def issue_g(i, p):
        (si, di, pr, ebuf, _, sg, _) = B[p]
        base = (wid + NW * i) * CB
        pltpu.sync_copy(sidx_hbm.at[pl.ds(base, CB)], si)
        pltpu.sync_copy(didx_hbm.at[pl.ds(base, CB)], di)
        pltpu.async_copy(proj.at[si], pr, sg)
        pltpu.async_copy(ep.at[pl.ds(base, CB)], ebuf, sg)

    def wait_g(i, p):
        (si, di, pr, ebuf, _, sg, _) = B[p]
        base = (wid + NW * i) * CB
        pltpu.make_async_copy(proj.at[si], pr, sg).wait()
        pltpu.make_async_copy(ep.at[pl.ds(base, CB)], ebuf, sg).wait()

    def start_s(p):
        (si, di, pr, ebuf, ch, _, ss) = B[p]
        pltpu.async_copy(ch, agg_sh.at[di], ss, add=True)

    def wait_s(p):
        (si, di, pr, ebuf, ch, _, ss) = B[p]
        pltpu.make_async_copy(ch, agg_sh.at[di], ss).wait()

    def compute(p):
        (_, _, pr, ebuf, ch, _, _) = B[p]

        def edge(e, ecarry):
            for j in range(H2 // 16):
                sl = pl.ds(j * 16, 16)
                pa, pb = plsc.unpack(plsc.bitcast(pr[e, sl], BF16),
                                     format=plsc.PackFormat.INTERLEAVED)
                ea, eb = plsc.unpack(plsc.bitcast(ebuf[e, sl], BF16),
                                     format=plsc.PackFormat.INTERLEAVED)
                xa = pa + ea
                xb = pb + eb
                ch[e, pl.ds(2 * j * 16, 16)] = jnp.maximum(xa, 0.1 * xa)
                ch[e, pl.ds((2 * j + 1) * 16, 16)] = jnp.maximum(xb, 0.1 * xb)
            return ecarry
        lax.fori_loop(0, CB, edge, 0)

    pl.when(nch > 0)(lambda: issue_g(0, 0))
    pl.when(nch > 1)(lambda: issue_g(1, 1))

    def pair(g, carry):
        i0 = 2 * g

        def b0():
            wait_g(i0, 0)
            compute(0)
            start_s(0)
        pl.when(i0 < nch)(b0)

        def a0():
            wait_s(0)
            issue_g(i0 + 2, 0)
        pl.when(i0 + 2 < nch)(a0)

        def b1():
            wait_g(i0 + 1, 1)
            compute(1)
            start_s(1)
        pl.when(i0 + 1 < nch)(b1)

        def a1():
            wait_s(1)
            issue_g(i0 + 3, 1)
        pl.when(i0 + 3 < nch)(a1)
        return carry
    lax.fori_loop(0, (nch + 1) // 2, pair, 0)
    pl.when(nch >= 1)(lambda: wait_s(0))
    pl.when(nch >= 2)(lambda: wait_s(1))
    plsc.subcore_barrier()

    def wb(i, carry):
        b = jnp.minimum((s + NSUB * i) * CB, N_MAIN - CB)
        pltpu.sync_copy(agg_sh.at[pl.ds(b, CB)], aggp.at[c, pl.ds(b, CB)])
        return carry
    lax.fori_loop(0, nz, wb, 0)


def _sc_wln(sidx, didx, proj, ep):
    buf = [
        pltpu.VMEM((CB,), jnp.int32),
        pltpu.VMEM((CB,), jnp.int32),
        pltpu.VMEM((CB, H2), jnp.int32),
        pltpu.VMEM((CB, H2), jnp.int32),
        pltpu.VMEM((CB, H), F32),
        pltpu.SemaphoreType.DMA,
        pltpu.SemaphoreType.DMA,
    ]
    return pl.kernel(
        _sc_wln_body,
        out_type=jax.ShapeDtypeStruct((NCORES, N_MAIN, H), F32),
        mesh=_mesh(),
        compiler_params=pltpu.CompilerParams(use_tc_tiling_on_sc=False,
                                             needs_layout_passes=False),
        scratch_types=[pltpu.VMEM_SHARED((N_MAIN, H), F32)] + buf + buf,
    )(sidx, didx, proj, ep)


def _gru(x, h, wihT, bih, whhT, bhh):
    gi = jnp.dot(x, wihT, preferred_element_type=F32) + bih
    gh = jnp.dot(h, whhT, preferred_element_type=F32) + bhh
    r = jax.nn.sigmoid(gi[:, :H] + gh[:, :H])
    z = jax.nn.sigmoid(gi[:, H:2 * H] + gh[:, H:2 * H])
    n = jnp.tanh(gi[:, 2 * H:] + r * gh[:, 2 * H:])
    return (1.0 - z) * n + z * h


def _post_main_body(mf, num2p, aggp, ts2mT, bts2m, u1aT, u1bT, bu1,
                    zm1T, bzm1, zm2T, bzm2, wihT, bih, whhT, bbhh, out):
    x = mf[...]
    sp = num2p[...]
    sp = sp[0] + sp[1]
    k_s2m = sp[:, :H] / (sp[:, H:H + 1] + 1e-6)
    stm = jnp.tanh(jnp.dot(k_s2m, ts2mT[...], preferred_element_type=F32)
                   + bts2m[...])
    ag = aggp[...]
    agg = ag[0] + ag[1]
    main_gnn = (jnp.dot(x, u1aT[...], preferred_element_type=F32)
                + jnp.dot(agg, u1bT[...], preferred_element_type=F32) + bu1[...])
    z = jax.nn.sigmoid(jnp.dot(main_gnn, zm1T[...], preferred_element_type=F32)
                       + bzm1[...]
                       + jnp.dot(stm, zm2T[...], preferred_element_type=F32)
                       + bzm2[...])
    h = (1.0 - z) * main_gnn + z * stm
    out[...] = _gru(h, x, wihT[...], bih[...], whhT[...], bbhh[...])


def _post_supe_body(sf, num1p, wsupeT, bwsupe, tm2sT, btm2s,
                    zs1T, bzs1, zs2T, bzs2, wihT, bih, whhT, bbhh, out):
    x = sf[...]
    sp = num1p[...]
    sp = sp[0] + sp[1]
    k_m2s = sp[:, :H] / (sp[:, H:H + 1] + 1e-6)
    mts = jnp.tanh(jnp.dot(k_m2s, tm2sT[...], preferred_element_type=F32)
                   + btm2s[...])
    self_supe = jnp.tanh(jnp.dot(x, wsupeT[...], preferred_element_type=F32)
                         + bwsupe[...])
    z = jax.nn.sigmoid(jnp.dot(self_supe, zs1T[...], preferred_element_type=F32)
                       + bzs1[...]
                       + jnp.dot(mts, zs2T[...], preferred_element_type=F32)
                       + bzs2[...])
    h = (1.0 - z) * self_supe + z * mts
    out[...] = _gru(h, x, wihT[...], bih[...], whhT[...], bbhh[...])


def _tc_post_main(mf, num2p, aggp, *weights):
    nb = 5
    blk = N_MAIN // nb
    specs = [pl.BlockSpec((blk, H), lambda i: (i, 0)),
             pl.BlockSpec((NCORES, blk, WD), lambda i: (0, i, 0)),
             pl.BlockSpec((NCORES, blk, H), lambda i: (0, i, 0))]
    specs += [_full(w.shape) for w in weights]
    return pl.pallas_call(
        _post_main_body,
        grid=(nb,),
        in_specs=specs,
        out_specs=pl.BlockSpec((blk, H), lambda i: (i, 0)),
        out_shape=jax.ShapeDtypeStruct((N_MAIN, H), F32),
    )(mf, num2p, aggp, *weights)


def _tc_post_supe(sf, num1p, *weights):
    return pl.pallas_call(
        _post_supe_body,
        out_shape=jax.ShapeDtypeStruct((N_SUPE, H), F32),
    )(sf, num1p, *weights)


def kernel(main_feat, supe_feat, edge_index_main, edge_attr,
           whole_src, whole_dst, params):
    p = params

    def t(name):
        return p[name].T

    def b(name):
        return p[name].reshape(1, -1)

    def p32(x):
        return lax.bitcast_convert_type(
            x.reshape(x.shape[0], H2, 2), jnp.int32)

    a_main, m_p, t_m_w, proj = _tc_pre_main(
        main_feat, t('Wa_main_w'), b('Wa_main_b'),
        t('scm_main_w'), b('scm_main_b'),
        t('scs_supe_w'), b('scs_supe_b'), p['scs_attn_w'].reshape(1, H),
        t('wln_u2_w')[:H, :], b('wln_u2_b'))
    a_supe, s_m_w, s_p = _tc_pre_supe(
        supe_feat, t('Wa_supe_w'), b('Wa_supe_b'),
        t('scm_supe_w'), b('scm_supe_b'),
        t('scs_main_w'), b('scs_main_b'), p['scm_attn_w'].reshape(1, H))
    ep = _tc_ep(edge_attr, t('wln_u2_w')[H:, :])

    out1 = _sc_attn_side(N_SUPE, 160, whole_dst, whole_src,
                         p32(s_m_w), p32(a_main), p32(m_p))
    out2 = _sc_attn_side(N_MAIN, 40, whole_src, whole_dst,
                         p32(t_m_w), p32(a_supe), p32(s_p))
    aggp = _sc_wln(edge_index_main[0], edge_index_main[1], p32(proj), p32(ep))

    main_out = _tc_post_main(
        main_feat, out2, aggp,
        t('t_s2m_w')[_PERM, :], b('t_s2m_b'),
        t('wln_u1_w')[:H, :], t('wln_u1_w')[H:, :][_PERM, :], b('wln_u1_b'),
        t('zm1_w'), b('zm1_b'), t('zm2_w'), b('zm2_b'),
        t('grum_wih'), b('grum_bih'), t('grum_whh'), b('grum_bhh'))
    supe_out = _tc_post_supe(
        supe_feat, out1,
        t('wsupe_w'), b('wsupe_b'), t('t_m2s_w')[_PERM, :], b('t_m2s_b'),
        t('zs1_w'), b('zs1_b'), t('zs2_w'), b('zs2_b'),
        t('grus_wih'), b('grus_bih'), t('grus_whh'), b('grus_bhh'))
    return jnp.concatenate([main_out, supe_out], axis=0)


# speedup vs baseline: 2.8610x; 2.8610x over previous
import functools

import jax
import jax.numpy as jnp
from jax import lax
from jax.experimental import pallas as pl
from jax.experimental.pallas import tpu as pltpu
from jax.experimental.pallas import tpu_sc as plsc

F32 = jnp.float32
H = 128
DE = 16
N_MAIN = 10000
N_SUPE = 1000
E_MAIN = 320000
E_WHOLE = 40000
NCORES = 2
NSUB = 16
NW = NCORES * NSUB
CB = 80
WD = 144
NCH_B = E_MAIN // CB


def _mesh():
    return plsc.VectorSubcoreMesh(core_axis_name="c", subcore_axis_name="s",
                                  num_cores=NCORES, num_subcores=NSUB)


def _pre_main_body(mf, waT, ba, scmT, bscm, scsT, bscs, attw, w2aT, bu2,
                   a_out, mp_out, tmw_out, proj_out):
    x = mf[...]
    a = jnp.tanh(jnp.dot(x, waT[...], preferred_element_type=F32) + ba[...])
    a_out[...] = a
    mp_out[...] = jnp.dot(a, scmT[...], preferred_element_type=F32) + bscm[...]
    tmw_out[...] = (jnp.dot(a, scsT[...], preferred_element_type=F32)
                    + bscs[...]) * attw[...]
    proj_out[...] = jnp.dot(x, w2aT[...], preferred_element_type=F32) + bu2[...]


def _pre_supe_body(sf, waT, ba, scmT, bscm, scsT, bscs, attw,
                   a_out, smw_out, sp_out):
    a = jnp.tanh(jnp.dot(sf[...], waT[...], preferred_element_type=F32) + ba[...])
    a_out[...] = a
    smw_out[...] = (jnp.dot(a, scmT[...], preferred_element_type=F32)
                    + bscm[...]) * attw[...]
    sp_out[...] = jnp.dot(a, scsT[...], preferred_element_type=F32) + bscs[...]


def _ep_body(ea, w2bT, out):
    out[...] = jnp.dot(ea[...], w2bT[...], preferred_element_type=F32)


def _full(shape):
    return pl.BlockSpec(shape, lambda i: tuple(0 for _ in shape))


def _tc_pre_main(mf, waT, ba, scmT, bscm, scsT, bscs, attw, w2aT, bu2):
    nb = 5
    blk = N_MAIN // nb
    row = pl.BlockSpec((blk, H), lambda i: (i, 0))
    w = _full((H, H))
    b = _full((1, H))
    out = jax.ShapeDtypeStruct((N_MAIN, H), F32)
    return pl.pallas_call(
        _pre_main_body,
        grid=(nb,),
        in_specs=[row, w, b, w, b, w, b, b, w, b],
        out_specs=[row, row, row, row],
        out_shape=[out, out, out, out],
    )(mf, waT, ba, scmT, bscm, scsT, bscs, attw, w2aT, bu2)


def _tc_pre_supe(sf, waT, ba, scmT, bscm, scsT, bscs, attw):
    out = jax.ShapeDtypeStruct((N_SUPE, H), F32)
    return pl.pallas_call(
        _pre_supe_body,
        out_shape=[out, out, out],
    )(sf, waT, ba, scmT, bscm, scsT, bscs, attw)


def _tc_ep(ea, w2bT):
    nb = 40
    blk = E_MAIN // nb
    return pl.pallas_call(
        _ep_body,
        grid=(nb,),
        in_specs=[pl.BlockSpec((blk, DE), lambda i: (i, 0)), _full((DE, H))],
        out_specs=pl.BlockSpec((blk, H), lambda i: (i, 0)),
        out_shape=jax.ShapeDtypeStruct((E_MAIN, H), F32),
    )(ea, w2bT)


def _sc_attn_body(nseg, ca, seg_hbm, oth_hbm, w_tab, x_tab, v_tab,
                  outp, num_sh, *bufs):
    (seg0, oth0, bw0, bx0, bv0, ch0, sg0, ss0,
     seg1, oth1, bw1, bx1, bv1, ch1, sg1, ss1) = bufs
    B = ((seg0, oth0, bw0, bx0, bv0, ch0, sg0, ss0),
         (seg1, oth1, bw1, bx1, bv1, ch1, sg1, ss1))
    c = lax.axis_index("c")
    s = lax.axis_index("s")
    wid = s * NCORES + c
    nch_tot = E_WHOLE // ca

    def zrow(r, carry):
        for j in range(WD // 16):
            ch0[r, pl.ds(j * 16, 16)] = jnp.zeros((16,), F32)
        return carry
    lax.fori_loop(0, min(ca, 64), zrow, 0)

    zc = min(ca, 64)
    nzc = ((nseg + zc - 1) // zc - s + NSUB - 1) // NSUB

    def z(i, carry):
        b = jnp.minimum((s + NSUB * i) * zc, nseg - zc)
        pltpu.sync_copy(ch0.at[pl.ds(0, zc)], num_sh.at[pl.ds(b, zc)])
        return carry
    lax.fori_loop(0, nzc, z, 0)
    plsc.subcore_barrier()

    lane = lax.iota(jnp.int32, 16)
    m0 = lane == 0
    perms = [(lane ^ k).reshape(16, 1) for k in (8, 4, 2, 1)]
    gdn = lax.GatherDimensionNumbers(
        offset_dims=(), collapsed_slice_dims=(0,), start_index_map=(0,))

    def allsum(v):
        for pm in perms:
            v = v + lax.gather(v, pm, gdn, (1,),
                               mode=lax.GatherScatterMode.PROMISE_IN_BOUNDS)
        return v

    nch = (nch_tot - wid + NW - 1) // NW

    def issue_g(i, p):
        (sv, ov, bw, bx, bv, _, sg, _) = B[p]
        base = (wid + NW * i) * ca
        pltpu.sync_copy(seg_hbm.at[pl.ds(base, ca)], sv)
        pltpu.sync_copy(oth_hbm.at[pl.ds(base, ca)], ov)
        pltpu.async_copy(w_tab.at[sv], bw, sg)
        pltpu.async_copy(x_tab.at[ov], bx, sg)
        pltpu.async_copy(v_tab.at[ov], bv, sg)

    def wait_g(p):
        (sv, ov, bw, bx, bv, _, sg, _) = B[p]
        pltpu.make_async_copy(w_tab.at[sv], bw, sg).wait()
        pltpu.make_async_copy(x_tab.at[ov], bx, sg).wait()
        pltpu.make_async_copy(v_tab.at[ov], bv, sg).wait()

    def start_s(p):
        (sv, _, _, _, _, ch, _, ss) = B[p]
        pltpu.async_copy(ch, num_sh.at[sv], ss, add=True)

    def wait_s(p):
        (sv, _, _, _, _, ch, _, ss) = B[p]
        pltpu.make_async_copy(ch, num_sh.at[sv], ss).wait()

    def compute(p):
        (_, _, bw, bx, bv, ch, _, _) = B[p]

        def edge(e, ecarry):
            acc = jnp.zeros((16,), F32)
            for j in range(H // 16):
                sl = pl.ds(j * 16, 16)
                acc = acc + bw[e, sl] * bx[e, sl]
            ev = jnp.exp(allsum(acc))
            for j in range(H // 16):
                sl = pl.ds(j * 16, 16)
                ch[e, sl] = ev * bv[e, sl]
            ch[e, pl.ds(H, 16)] = jnp.where(m0, ev, 0.0)
            return ecarry
        lax.fori_loop(0, ca, edge, 0)

    pl.when(nch > 0)(lambda: issue_g(0, 0))
    pl.when(nch > 1)(lambda: issue_g(1, 1))

    def pair(g, carry):
        i0 = 2 * g

        def b0():
            wait_g(0)
            compute(0)
            start_s(0)
        pl.when(i0 < nch)(b0)

        def a0():
            wait_s(0)
            issue_g(i0 + 2, 0)
        pl.when(i0 + 2 < nch)(a0)

        def b1():
            wait_g(1)
            compute(1)
            start_s(1)
        pl.when(i0 + 1 < nch)(b1)

        def a1():
            wait_s(1)
            issue_g(i0 + 3, 1)
        pl.when(i0 + 3 < nch)(a1)
        return carry
    lax.fori_loop(0, (nch + 1) // 2, pair, 0)
    pl.when(nch >= 1)(lambda: wait_s(0))
    pl.when(nch >= 2)(lambda: wait_s(1))
    plsc.subcore_barrier()

    def wb(i, carry):
        b = jnp.minimum((s + NSUB * i) * zc, nseg - zc)
        pltpu.sync_copy(num_sh.at[pl.ds(b, zc)], outp.at[c, pl.ds(b, zc)])
        return carry
    lax.fori_loop(0, nzc, wb, 0)


def _sc_attn_side(nseg, ca, seg_idx, oth_idx, w_tab, x_tab, v_tab):
    buf = [
        pltpu.VMEM((ca,), jnp.int32),
        pltpu.VMEM((ca,), jnp.int32),
        pltpu.VMEM((ca, H), F32),
        pltpu.VMEM((ca, H), F32),
        pltpu.VMEM((ca, H), F32),
        pltpu.VMEM((ca, WD), F32),
        pltpu.SemaphoreType.DMA,
        pltpu.SemaphoreType.DMA,
    ]
    return pl.kernel(
        functools.partial(_sc_attn_body, nseg, ca),
        out_type=jax.ShapeDtypeStruct((NCORES, nseg, WD), F32),
        mesh=_mesh(),
        compiler_params=pltpu.CompilerParams(use_tc_tiling_on_sc=False),
        scratch_types=[pltpu.VMEM_SHARED((nseg, WD), F32)] + buf + buf,
    )(seg_idx, oth_idx, w_tab, x_tab, v_tab)


def _sc_wln_body(sidx_hbm, didx_hbm, proj, ep, aggp, agg_sh, *bufs):
    (si0, di0, pr0, ep0, sg0, ss0,
     si1, di1, pr1, ep1, sg1, ss1) = bufs
    B = ((si0, di0, pr0, ep0, sg0, ss0),
         (si1, di1, pr1, ep1, sg1, ss1))
    c = lax.axis_index("c")
    s = lax.axis_index("s")
    wid = s * NCORES + c

    def zrow(r, carry):
        for j in range(H // 16):
            pr0[r, pl.ds(j * 16, 16)] = jnp.zeros((16,), F32)
        return carry
    lax.fori_loop(0, CB, zrow, 0)

    nz = ((N_MAIN + CB - 1) // CB - s + NSUB - 1) // NSUB

    def z(i, carry):
        b = jnp.minimum((s + NSUB * i) * CB, N_MAIN - CB)
        pltpu.sync_copy(pr0, agg_sh.at[pl.ds(b, CB)])
        return carry
    lax.fori_loop(0, nz, z, 0)
    plsc.subcore_barrier()

    nch = (NCH_B - wid + NW - 1) // NW

    def issue_g(i, p):
        (si, di, pr, ebuf, sg, _) = B[p]
        base = (wid + NW * i) * CB
        pltpu.sync_copy(sidx_hbm.at[pl.ds(base, CB)], si)
        pltpu.sync_copy(didx_hbm.at[pl.ds(base, CB)], di)
        pltpu.async_copy(proj.at[si], pr, sg)
        pltpu.async_copy(ep.at[pl.ds(base, CB)], ebuf, sg)

    def wait_g(i, p):
        (si, di, pr, ebuf, sg, _) = B[p]
        base = (wid + NW * i) * CB
        pltpu.make_async_copy(proj.at[si], pr, sg).wait()
        pltpu.make_async_copy(ep.at[pl.ds(base, CB)], ebuf, sg).wait()

    def start_s(p):
        (si, di, pr, ebuf, _, ss) = B[p]
        pltpu.async_copy(pr, agg_sh.at[di], ss, add=True)

    def wait_s(p):
        (si, di, pr, ebuf, _, ss) = B[p]
        pltpu.make_async_copy(pr, agg_sh.at[di], ss).wait()

    def compute(p):
        (_, _, pr, ebuf, _, _) = B[p]

        def edge(e, ecarry):
            for j in range(H // 16):
                sl = pl.ds(j * 16, 16)
                x = pr[e, sl] + ebuf[e, sl]
                pr[e, sl] = jnp.maximum(x, 0.1 * x)
            return ecarry
        lax.fori_loop(0, CB, edge, 0)

    pl.when(nch > 0)(lambda: issue_g(0, 0))
    pl.when(nch > 1)(lambda: issue_g(1, 1))

    def pair(g, carry):
        i0 = 2 * g

        def b0():
            wait_g(i0, 0)
            compute(0)
            start_s(0)
        pl.when(i0 < nch)(b0)

        def a0():
            wait_s(0)
            issue_g(i0 + 2, 0)
        pl.when(i0 + 2 < nch)(a0)

        def b1():
            wait_g(i0 + 1, 1)
            compute(1)
            start_s(1)
        pl.when(i0 + 1 < nch)(b1)

        def a1():
            wait_s(1)
            issue_g(i0 + 3, 1)
        pl.when(i0 + 3 < nch)(a1)
        return carry
    lax.fori_loop(0, (nch + 1) // 2, pair, 0)
    pl.when(nch >= 1)(lambda: wait_s(0))
    pl.when(nch >= 2)(lambda: wait_s(1))
    plsc.subcore_barrier()

    def wb(i, carry):
        b = jnp.minimum((s + NSUB * i) * CB, N_MAIN - CB)
        pltpu.sync_copy(agg_sh.at[pl.ds(b, CB)], aggp.at[c, pl.ds(b, CB)])
        return carry
    lax.fori_loop(0, nz, wb, 0)


def _sc_wln(sidx, didx, proj, ep):
    buf = [
        pltpu.VMEM((CB,), jnp.int32),
        pltpu.VMEM((CB,), jnp.int32),
        pltpu.VMEM((CB, H), F32),
        pltpu.VMEM((CB, H), F32),
        pltpu.SemaphoreType.DMA,
        pltpu.SemaphoreType.DMA,
    ]
    return pl.kernel(
        _sc_wln_body,
        out_type=jax.ShapeDtypeStruct((NCORES, N_MAIN, H), F32),
        mesh=_mesh(),
        scratch_types=[pltpu.VMEM_SHARED((N_MAIN, H), F32)] + buf + buf,
    )(sidx, didx, proj, ep)


def _gru(x, h, wihT, bih, whhT, bhh):
    gi = jnp.dot(x, wihT, preferred_element_type=F32) + bih
    gh = jnp.dot(h, whhT, preferred_element_type=F32) + bhh
    r = jax.nn.sigmoid(gi[:, :H] + gh[:, :H])
    z = jax.nn.sigmoid(gi[:, H:2 * H] + gh[:, H:2 * H])
    n = jnp.tanh(gi[:, 2 * H:] + r * gh[:, 2 * H:])
    return (1.0 - z) * n + z * h


def _post_main_body(mf, num2p, aggp, ts2mT, bts2m, u1aT, u1bT, bu1,
                    zm1T, bzm1, zm2T, bzm2, wihT, bih, whhT, bbhh, out):
    x = mf[...]
    sp = num2p[...]
    sp = sp[0] + sp[1]
    k_s2m = sp[:, :H] / (sp[:, H:H + 1] + 1e-6)
    stm = jnp.tanh(jnp.dot(k_s2m, ts2mT[...], preferred_element_type=F32)
                   + bts2m[...])
    ag = aggp[...]
    agg = ag[0] + ag[1]
    main_gnn = (jnp.dot(x, u1aT[...], preferred_element_type=F32)
                + jnp.dot(agg, u1bT[...], preferred_element_type=F32) + bu1[...])
    z = jax.nn.sigmoid(jnp.dot(main_gnn, zm1T[...], preferred_element_type=F32)
                       + bzm1[...]
                       + jnp.dot(stm, zm2T[...], preferred_element_type=F32)
                       + bzm2[...])
    h = (1.0 - z) * main_gnn + z * stm
    out[...] = _gru(h, x, wihT[...], bih[...], whhT[...], bbhh[...])


def _post_supe_body(sf, num1p, wsupeT, bwsupe, tm2sT, btm2s,
                    zs1T, bzs1, zs2T, bzs2, wihT, bih, whhT, bbhh, out):
    x = sf[...]
    sp = num1p[...]
    sp = sp[0] + sp[1]
    k_m2s = sp[:, :H] / (sp[:, H:H + 1] + 1e-6)
    mts = jnp.tanh(jnp.dot(k_m2s, tm2sT[...], preferred_element_type=F32)
                   + btm2s[...])
    self_supe = jnp.tanh(jnp.dot(x, wsupeT[...], preferred_element_type=F32)
                         + bwsupe[...])
    z = jax.nn.sigmoid(jnp.dot(self_supe, zs1T[...], preferred_element_type=F32)
                       + bzs1[...]
                       + jnp.dot(mts, zs2T[...], preferred_element_type=F32)
                       + bzs2[...])
    h = (1.0 - z) * self_supe + z * mts
    out[...] = _gru(h, x, wihT[...], bih[...], whhT[...], bbhh[...])


def _tc_post_main(mf, num2p, aggp, *weights):
    nb = 5
    blk = N_MAIN // nb
    specs = [pl.BlockSpec((blk, H), lambda i: (i, 0)),
             pl.BlockSpec((NCORES, blk, WD), lambda i: (0, i, 0)),
             pl.BlockSpec((NCORES, blk, H), lambda i: (0, i, 0))]
    specs += [_full(w.shape) for w in weights]
    return pl.pallas_call(
        _post_main_body,
        grid=(nb,),
        in_specs=specs,
        out_specs=pl.BlockSpec((blk, H), lambda i: (i, 0)),
        out_shape=jax.ShapeDtypeStruct((N_MAIN, H), F32),
    )(mf, num2p, aggp, *weights)


def _tc_post_supe(sf, num1p, *weights):
    return pl.pallas_call(
        _post_supe_body,
        out_shape=jax.ShapeDtypeStruct((N_SUPE, H), F32),
    )(sf, num1p, *weights)


def kernel(main_feat, supe_feat, edge_index_main, edge_attr,
           whole_src, whole_dst, params):
    p = params

    def t(name):
        return p[name].T

    def b(name):
        return p[name].reshape(1, -1)

    a_main, m_p, t_m_w, proj = _tc_pre_main(
        main_feat, t('Wa_main_w'), b('Wa_main_b'),
        t('scm_main_w'), b('scm_main_b'),
        t('scs_supe_w'), b('scs_supe_b'), p['scs_attn_w'].reshape(1, H),
        t('wln_u2_w')[:H, :], b('wln_u2_b'))
    a_supe, s_m_w, s_p = _tc_pre_supe(
        supe_feat, t('Wa_supe_w'), b('Wa_supe_b'),
        t('scm_supe_w'), b('scm_supe_b'),
        t('scs_main_w'), b('scs_main_b'), p['scm_attn_w'].reshape(1, H))
    ep = _tc_ep(edge_attr, t('wln_u2_w')[H:, :])

    out1 = _sc_attn_side(N_SUPE, 80, whole_dst, whole_src, s_m_w, a_main, m_p)
    out2 = _sc_attn_side(N_MAIN, 32, whole_src, whole_dst, t_m_w, a_supe, s_p)
    aggp = _sc_wln(edge_index_main[0], edge_index_main[1], proj, ep)

    main_out = _tc_post_main(
        main_feat, out2, aggp,
        t('t_s2m_w'), b('t_s2m_b'),
        t('wln_u1_w')[:H, :], t('wln_u1_w')[H:, :], b('wln_u1_b'),
        t('zm1_w'), b('zm1_b'), t('zm2_w'), b('zm2_b'),
        t('grum_wih'), b('grum_bih'), t('grum_whh'), b('grum_bhh'))
    supe_out = _tc_post_supe(
        supe_feat, out1,
        t('wsupe_w'), b('wsupe_b'), t('t_m2s_w'), b('t_m2s_b'),
        t('zs1_w'), b('zs1_b'), t('zs2_w'), b('zs2_b'),
        t('grus_wih'), b('grus_bih'), t('grus_whh'), b('grus_bhh'))
    return jnp.concatenate([main_out, supe_out], axis=0)
